# baseline (speedup 1.0000x reference)
import functools

import jax
import jax.numpy as jnp
from jax import lax
from jax.experimental import pallas as pl
from jax.experimental.pallas import tpu as pltpu
from jax.experimental.pallas import tpu_sc as plsc

_LANES = 16
_NC = 2
_NS = 16
_NW = _NC * _NS
_IDX_CHUNK = 128


def _make_center_loss(batch, feat):
    b_per_w = batch // _NW
    n_chunks = b_per_w // _IDX_CHUNK
    mesh = plsc.VectorSubcoreMesh(core_axis_name="c", subcore_axis_name="s")

    @functools.partial(
        pl.kernel,
        mesh=mesh,
        compiler_params=pltpu.CompilerParams(use_tc_tiling_on_sc=False),
        out_type=jax.ShapeDtypeStruct((_NW, _LANES), jnp.float32),
        scratch_types=[
            pltpu.VMEM((n_chunks, _IDX_CHUNK), jnp.int32),
            pltpu.VMEM((b_per_w, feat), jnp.float32),
            pltpu.VMEM((b_per_w, feat), jnp.float32),
            pltpu.VMEM((_LANES,), jnp.float32),
            pltpu.SemaphoreType.DMA,
            pltpu.SemaphoreType.DMA,
        ],
    )
    def center_loss(feat_hbm, lab_hbm, cent_hbm, out_hbm,
                    idx_v, feats_v, rows_v, acc_v, gsem, fsem):
        wid = lax.axis_index("s") * _NC + lax.axis_index("c")
        base = wid * b_per_w

        pltpu.sync_copy(lab_hbm.at[wid], idx_v)
        fcp = pltpu.async_copy(feat_hbm.at[pl.ds(base, b_per_w)], feats_v, fsem)
        gcps = [
            pltpu.async_copy(
                cent_hbm.at[idx_v.at[k]],
                rows_v.at[pl.ds(k * _IDX_CHUNK, _IDX_CHUNK)],
                gsem,
            )
            for k in range(n_chunks)
        ]
        fcp.wait()
        for gcp in gcps:
            gcp.wait()

        n_half = feat // _LANES

        def body(r, accs):
            out = []
            for h in range(n_half):
                f = feats_v[r, pl.ds(h * _LANES, _LANES)]
                c = rows_v[r, pl.ds(h * _LANES, _LANES)]
                d = f - c
                out.append(accs[h] + d * d)
            return tuple(out)

        zero = jnp.zeros((_LANES,), jnp.float32)
        accs = lax.fori_loop(0, b_per_w, body, (zero,) * n_half)
        total = accs[0]
        for h in range(1, n_half):
            total = total + accs[h]
        acc_v[...] = total
        pltpu.sync_copy(acc_v, out_hbm.at[wid])

    return center_loss


@jax.jit
def kernel(features, labels, centers):
    batch, feat = features.shape
    lab = labels.astype(jnp.int32).reshape(_NW, batch // (_NW * _IDX_CHUNK),
                                           _IDX_CHUNK)
    partials = _make_center_loss(batch, feat)(features, lab, centers)
    return jnp.sum(partials) / batch
